# Initial kernel scaffold; baseline (speedup 1.0000x reference)
import jax
import jax.numpy as jnp
from jax.experimental import pallas as pl


def kernel(inputs, embedding):
    raise NotImplementedError("write your pallas kernel here")


# speedup vs baseline: 4.1417x; 4.1417x over previous
import functools

import jax
import jax.numpy as jnp
from jax import lax
from jax.experimental import pallas as pl
from jax.experimental.pallas import tpu as pltpu
from jax.experimental.pallas import tpu_sc as plsc

BATCH = 4096
HIST = 200
EMBED_DIM = 64
B = BATCH * HIST

_info = plsc.get_sparse_core_info()
_NC, _NS = _info.num_cores, _info.num_subcores
NW = _NC * _NS
B_PER_W = B // NW
CHUNK = 1024
N_CHUNKS = B_PER_W // CHUNK

_mesh = plsc.VectorSubcoreMesh(core_axis_name="c", subcore_axis_name="s")


@functools.partial(
    pl.kernel,
    mesh=_mesh,
    out_type=jax.ShapeDtypeStruct((B, EMBED_DIM), jnp.float32),
    scratch_types=[
        pltpu.VMEM((CHUNK,), jnp.int32),
        pltpu.VMEM((CHUNK, EMBED_DIM), jnp.float32),
        pltpu.SemaphoreType.DMA,
    ],
    compiler_params=pltpu.CompilerParams(use_tc_tiling_on_sc=False),
)
def _gather_kernel(idx_hbm, table_hbm, out_hbm, idx_v, rows_v, sem):
    wid = lax.axis_index("s") * _NC + lax.axis_index("c")
    base = wid * B_PER_W

    def step(i, carry):
        off = base + i * CHUNK
        pltpu.sync_copy(idx_hbm.at[pl.ds(off, CHUNK)], idx_v)
        pltpu.async_copy(table_hbm.at[idx_v], rows_v, sem).wait()
        pltpu.sync_copy(rows_v, out_hbm.at[pl.ds(off, CHUNK)])
        return carry

    lax.fori_loop(0, N_CHUNKS, step, 0)


def kernel(inputs, embedding):
    idx = inputs.reshape(B).astype(jnp.int32)
    out = _gather_kernel(idx, embedding)
    return out.reshape(BATCH, HIST, EMBED_DIM)


# speedup vs baseline: 4.2549x; 1.0273x over previous
import functools

import jax
import jax.numpy as jnp
from jax import lax
from jax.experimental import pallas as pl
from jax.experimental.pallas import tpu as pltpu
from jax.experimental.pallas import tpu_sc as plsc

BATCH = 4096
HIST = 200
EMBED_DIM = 64
B = BATCH * HIST

_info = plsc.get_sparse_core_info()
_NC, _NS = _info.num_cores, _info.num_subcores
NW = _NC * _NS
B_PER_W = B // NW
CHUNK = 512
N_CHUNKS = B_PER_W // CHUNK
NBUF = 3

_mesh = plsc.VectorSubcoreMesh(core_axis_name="c", subcore_axis_name="s")


@functools.partial(
    pl.kernel,
    mesh=_mesh,
    out_type=jax.ShapeDtypeStruct((B, EMBED_DIM), jnp.float32),
    scratch_types=[
        pltpu.VMEM((B_PER_W,), jnp.int32),
        pltpu.VMEM((CHUNK, EMBED_DIM), jnp.float32),
        pltpu.VMEM((CHUNK, EMBED_DIM), jnp.float32),
        pltpu.VMEM((CHUNK, EMBED_DIM), jnp.float32),
        pltpu.SemaphoreType.DMA,
        pltpu.SemaphoreType.DMA,
        pltpu.SemaphoreType.DMA,
        pltpu.SemaphoreType.DMA,
        pltpu.SemaphoreType.DMA,
        pltpu.SemaphoreType.DMA,
    ],
    compiler_params=pltpu.CompilerParams(use_tc_tiling_on_sc=False),
)
def _gather_kernel(idx_hbm, table_hbm, out_hbm, idx_all,
                   rows0, rows1, rows2, sg0, sg1, sg2, sw0, sw1, sw2):
    wid = lax.axis_index("s") * _NC + lax.axis_index("c")
    base = wid * B_PER_W

    rows = (rows0, rows1, rows2)
    sg = (sg0, sg1, sg2)
    sw = (sw0, sw1, sw2)

    pltpu.sync_copy(idx_hbm.at[pl.ds(base, B_PER_W)], idx_all)

    def gather_desc(i, b):
        src = table_hbm.at[idx_all.at[pl.ds(i * CHUNK, CHUNK)]]
        return pltpu.make_async_copy(src, rows[b], sg[b])

    def wb_desc(i, b):
        dst = out_hbm.at[pl.ds(base + i * CHUNK, CHUNK)]
        return pltpu.make_async_copy(rows[b], dst, sw[b])


    gather_desc(0, 0).start()
    gather_desc(1, 1).start()
    gather_desc(0, 0).wait()
    wb_desc(0, 0).start()
    gather_desc(2, 2).start()
    gather_desc(1, 1).wait()
    wb_desc(1, 1).start()

    def group(g, carry):
        for b in range(NBUF):
            i = NBUF * g + b
            wb_desc(i - NBUF, b).wait()
            gather_desc(i, b).start()
            gather_desc(i - 1, (b - 1) % NBUF).wait()
            wb_desc(i - 1, (b - 1) % NBUF).start()
        return carry

    lax.fori_loop(1, (N_CHUNKS - 2) // NBUF, group, 0)

    for i in (N_CHUNKS - 2, N_CHUNKS - 1):
        b = i % NBUF
        wb_desc(i - NBUF, b).wait()
        gather_desc(i, b).start()
        gather_desc(i - 1, (b - 1) % NBUF).wait()
        wb_desc(i - 1, (b - 1) % NBUF).start()

    i = N_CHUNKS - 1
    gather_desc(i, i % NBUF).wait()
    wb_desc(i, i % NBUF).start()
    for j in (N_CHUNKS - 3, N_CHUNKS - 2, N_CHUNKS - 1):
        wb_desc(j, j % NBUF).wait()


def kernel(inputs, embedding):
    idx = inputs.reshape(B).astype(jnp.int32)
    out = _gather_kernel(idx, embedding)
    return out.reshape(BATCH, HIST, EMBED_DIM)


# speedup vs baseline: 4.2576x; 1.0006x over previous
import functools

import jax
import jax.numpy as jnp
from jax import lax
from jax.experimental import pallas as pl
from jax.experimental.pallas import tpu as pltpu
from jax.experimental.pallas import tpu_sc as plsc

BATCH = 4096
HIST = 200
EMBED_DIM = 64

_info = plsc.get_sparse_core_info()
_NC, _NS = _info.num_cores, _info.num_subcores
NW = _NC * _NS
ROWS_PER_W = BATCH // NW
NBUF = 3

_mesh = plsc.VectorSubcoreMesh(core_axis_name="c", subcore_axis_name="s")


@functools.partial(
    pl.kernel,
    mesh=_mesh,
    out_type=jax.ShapeDtypeStruct((BATCH, HIST, EMBED_DIM), jnp.float32),
    scratch_types=[
        pltpu.VMEM((ROWS_PER_W, HIST), jnp.int32),
        pltpu.VMEM((HIST, EMBED_DIM), jnp.float32),
        pltpu.VMEM((HIST, EMBED_DIM), jnp.float32),
        pltpu.VMEM((HIST, EMBED_DIM), jnp.float32),
        pltpu.SemaphoreType.DMA,
        pltpu.SemaphoreType.DMA,
        pltpu.SemaphoreType.DMA,
        pltpu.SemaphoreType.DMA,
        pltpu.SemaphoreType.DMA,
        pltpu.SemaphoreType.DMA,
    ],
    compiler_params=pltpu.CompilerParams(use_tc_tiling_on_sc=False),
)
def _gather_kernel(idx_hbm, table_hbm, out_hbm, idx_all,
                   rows0, rows1, rows2, sg0, sg1, sg2, sw0, sw1, sw2):
    wid = lax.axis_index("s") * _NC + lax.axis_index("c")
    row0 = wid * ROWS_PER_W

    rows = (rows0, rows1, rows2)
    sg = (sg0, sg1, sg2)
    sw = (sw0, sw1, sw2)

    pltpu.sync_copy(idx_hbm.at[pl.ds(row0, ROWS_PER_W)], idx_all)

    def gather_desc(i, b):
        src = table_hbm.at[idx_all.at[i]]
        return pltpu.make_async_copy(src, rows[b], sg[b])

    def wb_desc(i, b):
        return pltpu.make_async_copy(rows[b], out_hbm.at[row0 + i], sw[b])


    gather_desc(0, 0).start()
    gather_desc(1, 1).start()
    gather_desc(0, 0).wait()
    wb_desc(0, 0).start()
    gather_desc(2, 2).start()
    gather_desc(1, 1).wait()
    wb_desc(1, 1).start()

    def group(g, carry):
        for b in range(NBUF):
            i = NBUF * g + b
            wb_desc(i - NBUF, b).wait()
            gather_desc(i, b).start()
            gather_desc(i - 1, (b - 1) % NBUF).wait()
            wb_desc(i - 1, (b - 1) % NBUF).start()
        return carry

    lax.fori_loop(1, (ROWS_PER_W - 2) // NBUF, group, 0)

    for i in (ROWS_PER_W - 2, ROWS_PER_W - 1):
        b = i % NBUF
        wb_desc(i - NBUF, b).wait()
        gather_desc(i, b).start()
        gather_desc(i - 1, (b - 1) % NBUF).wait()
        wb_desc(i - 1, (b - 1) % NBUF).start()

    i = ROWS_PER_W - 1
    gather_desc(i, i % NBUF).wait()
    wb_desc(i, i % NBUF).start()
    for j in (ROWS_PER_W - 3, ROWS_PER_W - 2, ROWS_PER_W - 1):
        wb_desc(j, j % NBUF).wait()


def kernel(inputs, embedding):
    return _gather_kernel(inputs.astype(jnp.int32), embedding)
